# baseline (speedup 1.0000x reference)
import functools

import jax
import jax.numpy as jnp
from jax import lax
from jax.experimental import pallas as pl
from jax.experimental.pallas import tpu as pltpu
from jax.experimental.pallas import tpu_sc as plsc

B = 8
N = 2048
K = 16
L = 16
NWORK = 32
NB_SC = 8
NB_TC = B - NB_SC
QPW = (NB_SC * N) // NWORK
QGROUPS = QPW // L
CHUNKS = N // L
BIG = 3.0e38
BQ = 256


def _bcast(vec, lane):
    return vec.at[lane].get(mode="promise_in_bounds")


def _round_bf16(v):
    u = plsc.bitcast(v, jnp.int32)
    r = (u + 0x7FFF + ((u >> 16) & 1)) & ~0xFFFF
    return plsc.bitcast(r, jnp.float32)


def _knn_sc_body(x0h, x1h, x2h, outh, y0v, y1v, y2v,
                 y0b, y1b, y2b, yyv, obuf):
    wid = lax.axis_index("s") * 2 + lax.axis_index("c")
    wpb = NWORK // NB_SC
    b = wid // wpb
    qoff = (wid % wpb) * QPW

    pltpu.sync_copy(x0h.at[b], y0v)
    pltpu.sync_copy(x1h.at[b], y1v)
    pltpu.sync_copy(x2h.at[b], y2v)

    def setup_body(c, carry):
        a0 = y0v[pl.ds(c * L, L)]
        a1 = y1v[pl.ds(c * L, L)]
        a2 = y2v[pl.ds(c * L, L)]
        yyv[pl.ds(c * L, L)] = (a0 * a0 + a1 * a1) + a2 * a2
        y0b[pl.ds(c * L, L)] = _round_bf16(a0)
        y1b[pl.ds(c * L, L)] = _round_bf16(a1)
        y2b[pl.ds(c * L, L)] = _round_bf16(a2)
        return carry

    lax.fori_loop(0, CHUNKS, setup_body, 0)

    iota = lax.broadcasted_iota(jnp.int32, (L,), 0)
    fifteen = jnp.full((L,), L - 1, jnp.int32)
    CU = 4
    QU = 4

    def _merge(st, d, ci):
        tv, ti, _ = st
        sd, si = plsc.sort_key_val(d, ci)
        rd = lax.rev(sd, (0,))
        ri = lax.rev(si, (0,))
        lt = tv < rd
        eq = tv == rd
        pick = lt | (eq & (ti < ri))
        nv = jnp.where(pick, tv, rd)
        ni = jnp.where(pick, ti, ri)
        nv2, ni2 = plsc.sort_key_val(nv, ni)
        return nv2, ni2, _bcast(nv2, fifteen)

    def qgroup(g, carry0):
        qb = qoff + g * L
        q0 = y0b[pl.ds(qb, L)]
        q1 = y1b[pl.ds(qb, L)]
        q2 = y2b[pl.ds(qb, L)]
        qq = yyv[pl.ds(qb, L)]

        def lquad(lq, carry1):
            bx0 = []
            bx1 = []
            bx2 = []
            bxx = []
            for u in range(QU):
                lv = jnp.full((L,), lq * QU + u, jnp.int32)
                bx0.append(_bcast(q0, lv))
                bx1.append(_bcast(q1, lv))
                bx2.append(_bcast(q2, lv))
                bxx.append(_bcast(qq, lv))

            def block(blk, st):
                d = []
                for u in range(CU):
                    base = blk * (CU * L) + u * L
                    yc0 = y0b[pl.ds(base, L)]
                    yc1 = y1b[pl.ds(base, L)]
                    yc2 = y2b[pl.ds(base, L)]
                    yyc = yyv[pl.ds(base, L)]
                    dq = []
                    for q in range(QU):
                        t = (bx0[q] * yc0 + bx1[q] * yc1) + bx2[q] * yc2
                        dq.append((bxx[q] + (-2.0) * t) + yyc)
                    d.append(dq)
                hits = []
                anyhit = None
                for q in range(QU):
                    m = d[0][q]
                    for u in range(1, CU):
                        m = jnp.minimum(m, d[u][q])
                    h = m < st[3 * q + 2]
                    hits.append(h)
                    anyhit = h if anyhit is None else (anyhit | h)

                def fine(stf):
                    out = list(stf)
                    for q in range(QU):
                        def fineq(stq, q=q):
                            s = stq
                            for u in range(CU):
                                ci = iota + (blk * (CU * L) + u * L)
                                cu = plsc.all_reduce_population_count(
                                    d[u][q] < s[2])
                                s = lax.cond(
                                    cu[0] > 0,
                                    lambda s2, u=u, ci=ci, q=q:
                                        _merge(s2, d[u][q], ci),
                                    lambda s2: s2, s)
                            return s
                        trip = (out[3 * q], out[3 * q + 1], out[3 * q + 2])
                        cq = plsc.all_reduce_population_count(hits[q])
                        trip = lax.cond(cq[0] > 0, fineq,
                                        lambda s2: s2, trip)
                        out[3 * q] = trip[0]
                        out[3 * q + 1] = trip[1]
                        out[3 * q + 2] = trip[2]
                    return tuple(out)

                cnt = plsc.all_reduce_population_count(anyhit)
                return lax.cond(cnt[0] > 0, fine, lambda s2: s2, st)

            init = []
            for _ in range(QU):
                init += [jnp.full((L,), BIG, jnp.float32),
                         jnp.full((L,), N, jnp.int32),
                         jnp.full((L,), BIG, jnp.float32)]
            fin = lax.fori_loop(0, CHUNKS // CU, block, tuple(init))
            for u in range(QU):
                obuf[pl.ds((g * L + lq * QU + u) * L, L)] = fin[3 * u + 1]
            return carry1

        lax.fori_loop(0, L // QU, lquad, 0)
        return carry0

    lax.fori_loop(0, QGROUPS, qgroup, 0)
    pltpu.sync_copy(obuf, outh.at[pl.ds(wid * QPW * K, QPW * K)])


def _knn_sc(xyz_sc):
    x0 = xyz_sc[..., 0]
    x1 = xyz_sc[..., 1]
    x2 = xyz_sc[..., 2]
    mesh = plsc.VectorSubcoreMesh(core_axis_name="c", subcore_axis_name="s")
    flat = functools.partial(
        pl.kernel,
        mesh=mesh,
        compiler_params=pltpu.CompilerParams(needs_layout_passes=False),
        out_type=jax.ShapeDtypeStruct((NB_SC * N * K,), jnp.int32),
        scratch_types=[
            pltpu.VMEM((N,), jnp.float32),
            pltpu.VMEM((N,), jnp.float32),
            pltpu.VMEM((N,), jnp.float32),
            pltpu.VMEM((N,), jnp.float32),
            pltpu.VMEM((N,), jnp.float32),
            pltpu.VMEM((N,), jnp.float32),
            pltpu.VMEM((N,), jnp.float32),
            pltpu.VMEM((QPW * K,), jnp.int32),
        ],
    )(_knn_sc_body)
    return flat(x0, x1, x2).reshape(NB_SC, N, K)


def _knn_tc_body(x_ref, yt_ref, o_ref):
    x = x_ref[0]
    yt = yt_ref[0]
    xx = jnp.sum(x * x, axis=1, keepdims=True)
    yy = jnp.sum(yt * yt, axis=0, keepdims=True)
    inner = -2.0 * jax.lax.dot_general(
        x, yt, (((1,), (0,)), ((), ())),
        preferred_element_type=jnp.float32)
    d = (xx + inner) + yy
    col = jax.lax.broadcasted_iota(jnp.int32, (BQ, N), 1)
    cols = []
    for _ in range(K):
        j = jnp.argmin(d, axis=1).astype(jnp.int32)[:, None]
        cols.append(j)
        d = jnp.where(col == j, BIG, d)
    o_ref[0] = jnp.concatenate(cols, axis=1)


def _knn_tc(xyz_tc):
    nb = xyz_tc.shape[0]
    yt = jnp.transpose(xyz_tc, (0, 2, 1))
    return pl.pallas_call(
        _knn_tc_body,
        grid=(nb, N // BQ),
        in_specs=[
            pl.BlockSpec((1, BQ, 3), lambda b, q: (b, q, 0)),
            pl.BlockSpec((1, 3, N), lambda b, q: (b, 0, 0)),
        ],
        out_specs=pl.BlockSpec((1, BQ, K), lambda b, q: (b, q, 0)),
        out_shape=jax.ShapeDtypeStruct((nb, N, K), jnp.int32),
    )(xyz_tc, yt)


def kernel(xyz):
    out_sc = _knn_sc(xyz[NB_TC:])
    if NB_TC == 0:
        return out_sc
    out_tc = _knn_tc(xyz[:NB_TC])
    return jnp.concatenate([out_tc, out_sc], axis=0)


# speedup vs baseline: 3.3388x; 3.3388x over previous
import functools

import jax
import jax.numpy as jnp
from jax import lax
from jax.experimental import pallas as pl
from jax.experimental.pallas import tpu as pltpu
from jax.experimental.pallas import tpu_sc as plsc

B = 8
N = 2048
K = 16
L = 16
NWORK = 32
NB_SC = 1
NB_TC = B - NB_SC
QPW = (NB_SC * N) // NWORK
QGROUPS = QPW // L
CHUNKS = N // L
BIG = 3.0e38
BQ = 256


def _bcast(vec, lane):
    return vec.at[lane].get(mode="promise_in_bounds")


def _round_bf16(v):
    u = plsc.bitcast(v, jnp.int32)
    r = (u + 0x7FFF + ((u >> 16) & 1)) & ~0xFFFF
    return plsc.bitcast(r, jnp.float32)


def _knn_sc_body(x0h, x1h, x2h, outh, y0v, y1v, y2v,
                 y0b, y1b, y2b, yyv, obuf):
    wid = lax.axis_index("s") * 2 + lax.axis_index("c")
    wpb = NWORK // NB_SC
    b = wid // wpb
    qoff = (wid % wpb) * QPW

    pltpu.sync_copy(x0h.at[b], y0v)
    pltpu.sync_copy(x1h.at[b], y1v)
    pltpu.sync_copy(x2h.at[b], y2v)

    def setup_body(c, carry):
        a0 = y0v[pl.ds(c * L, L)]
        a1 = y1v[pl.ds(c * L, L)]
        a2 = y2v[pl.ds(c * L, L)]
        yyv[pl.ds(c * L, L)] = (a0 * a0 + a1 * a1) + a2 * a2
        y0b[pl.ds(c * L, L)] = _round_bf16(a0)
        y1b[pl.ds(c * L, L)] = _round_bf16(a1)
        y2b[pl.ds(c * L, L)] = _round_bf16(a2)
        return carry

    lax.fori_loop(0, CHUNKS, setup_body, 0)

    iota = lax.broadcasted_iota(jnp.int32, (L,), 0)
    fifteen = jnp.full((L,), L - 1, jnp.int32)
    CU = 4
    QU = 4

    def _merge(st, d, ci):
        tv, ti, _ = st
        sd, si = plsc.sort_key_val(d, ci)
        rd = lax.rev(sd, (0,))
        ri = lax.rev(si, (0,))
        lt = tv < rd
        eq = tv == rd
        pick = lt | (eq & (ti < ri))
        nv = jnp.where(pick, tv, rd)
        ni = jnp.where(pick, ti, ri)
        nv2, ni2 = plsc.sort_key_val(nv, ni)
        return nv2, ni2, _bcast(nv2, fifteen)

    def qgroup(g, carry0):
        qb = qoff + g * L
        q0 = y0b[pl.ds(qb, L)]
        q1 = y1b[pl.ds(qb, L)]
        q2 = y2b[pl.ds(qb, L)]
        qq = yyv[pl.ds(qb, L)]

        def lquad(lq, carry1):
            bx0 = []
            bx1 = []
            bx2 = []
            bxx = []
            for u in range(QU):
                lv = jnp.full((L,), lq * QU + u, jnp.int32)
                bx0.append(_bcast(q0, lv))
                bx1.append(_bcast(q1, lv))
                bx2.append(_bcast(q2, lv))
                bxx.append(_bcast(qq, lv))

            def block(blk, st):
                d = []
                for u in range(CU):
                    base = blk * (CU * L) + u * L
                    yc0 = y0b[pl.ds(base, L)]
                    yc1 = y1b[pl.ds(base, L)]
                    yc2 = y2b[pl.ds(base, L)]
                    yyc = yyv[pl.ds(base, L)]
                    dq = []
                    for q in range(QU):
                        t = (bx0[q] * yc0 + bx1[q] * yc1) + bx2[q] * yc2
                        dq.append((bxx[q] + (-2.0) * t) + yyc)
                    d.append(dq)
                hits = []
                anyhit = None
                for q in range(QU):
                    m = d[0][q]
                    for u in range(1, CU):
                        m = jnp.minimum(m, d[u][q])
                    h = m < st[3 * q + 2]
                    hits.append(h)
                    anyhit = h if anyhit is None else (anyhit | h)

                def fine(stf):
                    out = list(stf)
                    for q in range(QU):
                        def fineq(stq, q=q):
                            s = stq
                            for u in range(CU):
                                ci = iota + (blk * (CU * L) + u * L)
                                cu = plsc.all_reduce_population_count(
                                    d[u][q] < s[2])
                                s = lax.cond(
                                    cu[0] > 0,
                                    lambda s2, u=u, ci=ci, q=q:
                                        _merge(s2, d[u][q], ci),
                                    lambda s2: s2, s)
                            return s
                        trip = (out[3 * q], out[3 * q + 1], out[3 * q + 2])
                        cq = plsc.all_reduce_population_count(hits[q])
                        trip = lax.cond(cq[0] > 0, fineq,
                                        lambda s2: s2, trip)
                        out[3 * q] = trip[0]
                        out[3 * q + 1] = trip[1]
                        out[3 * q + 2] = trip[2]
                    return tuple(out)

                cnt = plsc.all_reduce_population_count(anyhit)
                return lax.cond(cnt[0] > 0, fine, lambda s2: s2, st)

            init = []
            for _ in range(QU):
                init += [jnp.full((L,), BIG, jnp.float32),
                         jnp.full((L,), N, jnp.int32),
                         jnp.full((L,), BIG, jnp.float32)]
            fin = lax.fori_loop(0, CHUNKS // CU, block, tuple(init))
            for u in range(QU):
                obuf[pl.ds((g * L + lq * QU + u) * L, L)] = fin[3 * u + 1]
            return carry1

        lax.fori_loop(0, L // QU, lquad, 0)
        return carry0

    lax.fori_loop(0, QGROUPS, qgroup, 0)
    pltpu.sync_copy(obuf, outh.at[pl.ds(wid * QPW * K, QPW * K)])


def _knn_sc(xyz_sc):
    x0 = xyz_sc[..., 0]
    x1 = xyz_sc[..., 1]
    x2 = xyz_sc[..., 2]
    mesh = plsc.VectorSubcoreMesh(core_axis_name="c", subcore_axis_name="s")
    flat = functools.partial(
        pl.kernel,
        mesh=mesh,
        compiler_params=pltpu.CompilerParams(needs_layout_passes=False),
        out_type=jax.ShapeDtypeStruct((NB_SC * N * K,), jnp.int32),
        scratch_types=[
            pltpu.VMEM((N,), jnp.float32),
            pltpu.VMEM((N,), jnp.float32),
            pltpu.VMEM((N,), jnp.float32),
            pltpu.VMEM((N,), jnp.float32),
            pltpu.VMEM((N,), jnp.float32),
            pltpu.VMEM((N,), jnp.float32),
            pltpu.VMEM((N,), jnp.float32),
            pltpu.VMEM((QPW * K,), jnp.int32),
        ],
    )(_knn_sc_body)
    return flat(x0, x1, x2).reshape(NB_SC, N, K)


def _knn_tc_body(x_ref, yt_ref, o_ref):
    x = x_ref[0]
    yt = yt_ref[0]
    xx = jnp.sum(x * x, axis=1, keepdims=True)
    yy = jnp.sum(yt * yt, axis=0, keepdims=True)
    inner = -2.0 * jax.lax.dot_general(
        x, yt, (((1,), (0,)), ((), ())),
        preferred_element_type=jnp.float32)
    d = (xx + inner) + yy
    H = N // 2
    dlo = d[:, :H]
    dhi = d[:, H:]
    side = dhi < dlo
    w = jnp.where(side, dhi, dlo)
    pmax = jnp.where(side, dlo, dhi)
    aux = side.astype(jnp.int32)
    col = jax.lax.broadcasted_iota(jnp.int32, (BQ, H), 1)
    cols = []
    for _ in range(K):
        j = jnp.argmin(w, axis=1).astype(jnp.int32)[:, None]
        onej = col == j
        a = jnp.min(jnp.where(onej, aux, 4), axis=1, keepdims=True)
        first = a < 2
        s = a & 1
        chosen = jnp.where(first, s, 1 - s)
        cols.append(j + H * chosen)
        w = jnp.where(onej, jnp.where(first, pmax, BIG), w)
        aux = jnp.where(onej, aux + 2, aux)
    o_ref[0] = jnp.concatenate(cols, axis=1)


def _knn_tc(xyz_tc):
    nb = xyz_tc.shape[0]
    yt = jnp.transpose(xyz_tc, (0, 2, 1))
    return pl.pallas_call(
        _knn_tc_body,
        grid=(nb, N // BQ),
        in_specs=[
            pl.BlockSpec((1, BQ, 3), lambda b, q: (b, q, 0)),
            pl.BlockSpec((1, 3, N), lambda b, q: (b, 0, 0)),
        ],
        out_specs=pl.BlockSpec((1, BQ, K), lambda b, q: (b, q, 0)),
        out_shape=jax.ShapeDtypeStruct((nb, N, K), jnp.int32),
    )(xyz_tc, yt)


def kernel(xyz):
    out_sc = _knn_sc(xyz[NB_TC:])
    if NB_TC == 0:
        return out_sc
    out_tc = _knn_tc(xyz[:NB_TC])
    return jnp.concatenate([out_tc, out_sc], axis=0)


# speedup vs baseline: 4.4737x; 1.3399x over previous
import functools

import jax
import jax.numpy as jnp
from jax import lax
from jax.experimental import pallas as pl
from jax.experimental.pallas import tpu as pltpu
from jax.experimental.pallas import tpu_sc as plsc

B = 8
N = 2048
K = 16
L = 16
NWORK = 32
NB_SC = 1
NB_TC = B - NB_SC
QPW = (NB_SC * N) // NWORK
QGROUPS = QPW // L
CHUNKS = N // L
BIG = 3.0e38
BQ = 256


def _bcast(vec, lane):
    return vec.at[lane].get(mode="promise_in_bounds")


def _round_bf16(v):
    u = plsc.bitcast(v, jnp.int32)
    r = (u + 0x7FFF + ((u >> 16) & 1)) & ~0xFFFF
    return plsc.bitcast(r, jnp.float32)


def _knn_sc_body(x0h, x1h, x2h, outh, y0v, y1v, y2v,
                 y0b, y1b, y2b, yyv, obuf):
    wid = lax.axis_index("s") * 2 + lax.axis_index("c")
    wpb = NWORK // NB_SC
    b = wid // wpb
    qoff = (wid % wpb) * QPW

    pltpu.sync_copy(x0h.at[b], y0v)
    pltpu.sync_copy(x1h.at[b], y1v)
    pltpu.sync_copy(x2h.at[b], y2v)

    def setup_body(c, carry):
        a0 = y0v[pl.ds(c * L, L)]
        a1 = y1v[pl.ds(c * L, L)]
        a2 = y2v[pl.ds(c * L, L)]
        yyv[pl.ds(c * L, L)] = (a0 * a0 + a1 * a1) + a2 * a2
        y0b[pl.ds(c * L, L)] = _round_bf16(a0)
        y1b[pl.ds(c * L, L)] = _round_bf16(a1)
        y2b[pl.ds(c * L, L)] = _round_bf16(a2)
        return carry

    lax.fori_loop(0, CHUNKS, setup_body, 0)

    iota = lax.broadcasted_iota(jnp.int32, (L,), 0)
    fifteen = jnp.full((L,), L - 1, jnp.int32)
    CU = 4
    QU = 4

    def _merge(st, d, ci):
        tv, ti, _ = st
        sd, si = plsc.sort_key_val(d, ci)
        rd = lax.rev(sd, (0,))
        ri = lax.rev(si, (0,))
        lt = tv < rd
        eq = tv == rd
        pick = lt | (eq & (ti < ri))
        nv = jnp.where(pick, tv, rd)
        ni = jnp.where(pick, ti, ri)
        nv2, ni2 = plsc.sort_key_val(nv, ni)
        return nv2, ni2, _bcast(nv2, fifteen)

    def qgroup(g, carry0):
        qb = qoff + g * L
        q0 = y0b[pl.ds(qb, L)]
        q1 = y1b[pl.ds(qb, L)]
        q2 = y2b[pl.ds(qb, L)]
        qq = yyv[pl.ds(qb, L)]

        def lquad(lq, carry1):
            bx0 = []
            bx1 = []
            bx2 = []
            bxx = []
            for u in range(QU):
                lv = jnp.full((L,), lq * QU + u, jnp.int32)
                bx0.append(_bcast(q0, lv))
                bx1.append(_bcast(q1, lv))
                bx2.append(_bcast(q2, lv))
                bxx.append(_bcast(qq, lv))

            def block(blk, st):
                d = []
                for u in range(CU):
                    base = blk * (CU * L) + u * L
                    yc0 = y0b[pl.ds(base, L)]
                    yc1 = y1b[pl.ds(base, L)]
                    yc2 = y2b[pl.ds(base, L)]
                    yyc = yyv[pl.ds(base, L)]
                    dq = []
                    for q in range(QU):
                        t = (bx0[q] * yc0 + bx1[q] * yc1) + bx2[q] * yc2
                        dq.append((bxx[q] + (-2.0) * t) + yyc)
                    d.append(dq)
                hits = []
                anyhit = None
                for q in range(QU):
                    m = d[0][q]
                    for u in range(1, CU):
                        m = jnp.minimum(m, d[u][q])
                    h = m < st[3 * q + 2]
                    hits.append(h)
                    anyhit = h if anyhit is None else (anyhit | h)

                def fine(stf):
                    out = list(stf)
                    for q in range(QU):
                        def fineq(stq, q=q):
                            s = stq
                            for u in range(CU):
                                ci = iota + (blk * (CU * L) + u * L)
                                cu = plsc.all_reduce_population_count(
                                    d[u][q] < s[2])
                                s = lax.cond(
                                    cu[0] > 0,
                                    lambda s2, u=u, ci=ci, q=q:
                                        _merge(s2, d[u][q], ci),
                                    lambda s2: s2, s)
                            return s
                        trip = (out[3 * q], out[3 * q + 1], out[3 * q + 2])
                        cq = plsc.all_reduce_population_count(hits[q])
                        trip = lax.cond(cq[0] > 0, fineq,
                                        lambda s2: s2, trip)
                        out[3 * q] = trip[0]
                        out[3 * q + 1] = trip[1]
                        out[3 * q + 2] = trip[2]
                    return tuple(out)

                cnt = plsc.all_reduce_population_count(anyhit)
                return lax.cond(cnt[0] > 0, fine, lambda s2: s2, st)

            init = []
            for _ in range(QU):
                init += [jnp.full((L,), BIG, jnp.float32),
                         jnp.full((L,), N, jnp.int32),
                         jnp.full((L,), BIG, jnp.float32)]
            fin = lax.fori_loop(0, CHUNKS // CU, block, tuple(init))
            for u in range(QU):
                obuf[pl.ds((g * L + lq * QU + u) * L, L)] = fin[3 * u + 1]
            return carry1

        lax.fori_loop(0, L // QU, lquad, 0)
        return carry0

    lax.fori_loop(0, QGROUPS, qgroup, 0)
    pltpu.sync_copy(obuf, outh.at[pl.ds(wid * QPW * K, QPW * K)])


def _knn_sc(xyz_sc):
    x0 = xyz_sc[..., 0]
    x1 = xyz_sc[..., 1]
    x2 = xyz_sc[..., 2]
    mesh = plsc.VectorSubcoreMesh(core_axis_name="c", subcore_axis_name="s")
    flat = functools.partial(
        pl.kernel,
        mesh=mesh,
        compiler_params=pltpu.CompilerParams(needs_layout_passes=False),
        out_type=jax.ShapeDtypeStruct((NB_SC * N * K,), jnp.int32),
        scratch_types=[
            pltpu.VMEM((N,), jnp.float32),
            pltpu.VMEM((N,), jnp.float32),
            pltpu.VMEM((N,), jnp.float32),
            pltpu.VMEM((N,), jnp.float32),
            pltpu.VMEM((N,), jnp.float32),
            pltpu.VMEM((N,), jnp.float32),
            pltpu.VMEM((N,), jnp.float32),
            pltpu.VMEM((QPW * K,), jnp.int32),
        ],
    )(_knn_sc_body)
    return flat(x0, x1, x2).reshape(NB_SC, N, K)


def _knn_tc_body(x_ref, yt_ref, o_ref):
    x = x_ref[0]
    yt = yt_ref[0]
    xx = jnp.sum(x * x, axis=1, keepdims=True)
    yy = jnp.sum(yt * yt, axis=0, keepdims=True)
    inner = -2.0 * jax.lax.dot_general(
        x, yt, (((1,), (0,)), ((), ())),
        preferred_element_type=jnp.float32)
    d = (xx + inner) + yy
    col = jax.lax.broadcasted_iota(jnp.int32, (BQ, N), 1)
    cols = []
    for _ in range(K):
        j = jnp.argmin(d, axis=1).astype(jnp.int32)[:, None]
        cols.append(j)
        d = jnp.where(col == j, BIG, d)
    o_ref[0] = jnp.concatenate(cols, axis=1)


def _knn_tc(xyz_tc):
    nb = xyz_tc.shape[0]
    yt = jnp.transpose(xyz_tc, (0, 2, 1))
    return pl.pallas_call(
        _knn_tc_body,
        grid=(nb, N // BQ),
        in_specs=[
            pl.BlockSpec((1, BQ, 3), lambda b, q: (b, q, 0)),
            pl.BlockSpec((1, 3, N), lambda b, q: (b, 0, 0)),
        ],
        out_specs=pl.BlockSpec((1, BQ, K), lambda b, q: (b, q, 0)),
        out_shape=jax.ShapeDtypeStruct((nb, N, K), jnp.int32),
    )(xyz_tc, yt)


def kernel(xyz):
    out_sc = _knn_sc(xyz[NB_TC:])
    if NB_TC == 0:
        return out_sc
    out_tc = _knn_tc(xyz[:NB_TC])
    return jnp.concatenate([out_tc, out_sc], axis=0)


# speedup vs baseline: 4.7275x; 1.0568x over previous
import functools

import jax
import jax.numpy as jnp
from jax import lax
from jax.experimental import pallas as pl
from jax.experimental.pallas import tpu as pltpu
from jax.experimental.pallas import tpu_sc as plsc

B = 8
N = 2048
K = 16
L = 16
NWORK = 32
NB_SC = 1
NB_TC = B - NB_SC
QPW = (NB_SC * N) // NWORK
QGROUPS = QPW // L
CHUNKS = N // L
BIG = 3.0e38
BQ = 512


def _bcast(vec, lane):
    return vec.at[lane].get(mode="promise_in_bounds")


def _round_bf16(v):
    u = plsc.bitcast(v, jnp.int32)
    r = (u + 0x7FFF + ((u >> 16) & 1)) & ~0xFFFF
    return plsc.bitcast(r, jnp.float32)


def _knn_sc_body(x0h, x1h, x2h, outh, y0v, y1v, y2v,
                 y0b, y1b, y2b, yyv, obuf):
    wid = lax.axis_index("s") * 2 + lax.axis_index("c")
    wpb = NWORK // NB_SC
    b = wid // wpb
    qoff = (wid % wpb) * QPW

    pltpu.sync_copy(x0h.at[b], y0v)
    pltpu.sync_copy(x1h.at[b], y1v)
    pltpu.sync_copy(x2h.at[b], y2v)

    def setup_body(c, carry):
        a0 = y0v[pl.ds(c * L, L)]
        a1 = y1v[pl.ds(c * L, L)]
        a2 = y2v[pl.ds(c * L, L)]
        yyv[pl.ds(c * L, L)] = (a0 * a0 + a1 * a1) + a2 * a2
        y0b[pl.ds(c * L, L)] = _round_bf16(a0)
        y1b[pl.ds(c * L, L)] = _round_bf16(a1)
        y2b[pl.ds(c * L, L)] = _round_bf16(a2)
        return carry

    lax.fori_loop(0, CHUNKS, setup_body, 0)

    iota = lax.broadcasted_iota(jnp.int32, (L,), 0)
    fifteen = jnp.full((L,), L - 1, jnp.int32)
    CU = 4
    QU = 4

    def _merge(st, d, ci):
        tv, ti, _ = st
        sd, si = plsc.sort_key_val(d, ci)
        rd = lax.rev(sd, (0,))
        ri = lax.rev(si, (0,))
        lt = tv < rd
        eq = tv == rd
        pick = lt | (eq & (ti < ri))
        nv = jnp.where(pick, tv, rd)
        ni = jnp.where(pick, ti, ri)
        nv2, ni2 = plsc.sort_key_val(nv, ni)
        return nv2, ni2, _bcast(nv2, fifteen)

    def qgroup(g, carry0):
        qb = qoff + g * L
        q0 = y0b[pl.ds(qb, L)]
        q1 = y1b[pl.ds(qb, L)]
        q2 = y2b[pl.ds(qb, L)]
        qq = yyv[pl.ds(qb, L)]

        def lquad(lq, carry1):
            bx0 = []
            bx1 = []
            bx2 = []
            bxx = []
            for u in range(QU):
                lv = jnp.full((L,), lq * QU + u, jnp.int32)
                bx0.append(_bcast(q0, lv))
                bx1.append(_bcast(q1, lv))
                bx2.append(_bcast(q2, lv))
                bxx.append(_bcast(qq, lv))

            def block(blk, st):
                d = []
                for u in range(CU):
                    base = blk * (CU * L) + u * L
                    yc0 = y0b[pl.ds(base, L)]
                    yc1 = y1b[pl.ds(base, L)]
                    yc2 = y2b[pl.ds(base, L)]
                    yyc = yyv[pl.ds(base, L)]
                    dq = []
                    for q in range(QU):
                        t = (bx0[q] * yc0 + bx1[q] * yc1) + bx2[q] * yc2
                        dq.append((bxx[q] + (-2.0) * t) + yyc)
                    d.append(dq)
                hits = []
                anyhit = None
                for q in range(QU):
                    m = d[0][q]
                    for u in range(1, CU):
                        m = jnp.minimum(m, d[u][q])
                    h = m < st[3 * q + 2]
                    hits.append(h)
                    anyhit = h if anyhit is None else (anyhit | h)

                def fine(stf):
                    out = list(stf)
                    for q in range(QU):
                        def fineq(stq, q=q):
                            s = stq
                            for u in range(CU):
                                ci = iota + (blk * (CU * L) + u * L)
                                cu = plsc.all_reduce_population_count(
                                    d[u][q] < s[2])
                                s = lax.cond(
                                    cu[0] > 0,
                                    lambda s2, u=u, ci=ci, q=q:
                                        _merge(s2, d[u][q], ci),
                                    lambda s2: s2, s)
                            return s
                        trip = (out[3 * q], out[3 * q + 1], out[3 * q + 2])
                        cq = plsc.all_reduce_population_count(hits[q])
                        trip = lax.cond(cq[0] > 0, fineq,
                                        lambda s2: s2, trip)
                        out[3 * q] = trip[0]
                        out[3 * q + 1] = trip[1]
                        out[3 * q + 2] = trip[2]
                    return tuple(out)

                cnt = plsc.all_reduce_population_count(anyhit)
                return lax.cond(cnt[0] > 0, fine, lambda s2: s2, st)

            init = []
            for _ in range(QU):
                init += [jnp.full((L,), BIG, jnp.float32),
                         jnp.full((L,), N, jnp.int32),
                         jnp.full((L,), BIG, jnp.float32)]
            fin = lax.fori_loop(0, CHUNKS // CU, block, tuple(init))
            for u in range(QU):
                obuf[pl.ds((g * L + lq * QU + u) * L, L)] = fin[3 * u + 1]
            return carry1

        lax.fori_loop(0, L // QU, lquad, 0)
        return carry0

    lax.fori_loop(0, QGROUPS, qgroup, 0)
    pltpu.sync_copy(obuf, outh.at[pl.ds(wid * QPW * K, QPW * K)])


def _knn_sc(xyz_sc):
    x0 = xyz_sc[..., 0]
    x1 = xyz_sc[..., 1]
    x2 = xyz_sc[..., 2]
    mesh = plsc.VectorSubcoreMesh(core_axis_name="c", subcore_axis_name="s")
    flat = functools.partial(
        pl.kernel,
        mesh=mesh,
        compiler_params=pltpu.CompilerParams(needs_layout_passes=False),
        out_type=jax.ShapeDtypeStruct((NB_SC * N * K,), jnp.int32),
        scratch_types=[
            pltpu.VMEM((N,), jnp.float32),
            pltpu.VMEM((N,), jnp.float32),
            pltpu.VMEM((N,), jnp.float32),
            pltpu.VMEM((N,), jnp.float32),
            pltpu.VMEM((N,), jnp.float32),
            pltpu.VMEM((N,), jnp.float32),
            pltpu.VMEM((N,), jnp.float32),
            pltpu.VMEM((QPW * K,), jnp.int32),
        ],
    )(_knn_sc_body)
    return flat(x0, x1, x2).reshape(NB_SC, N, K)


def _knn_tc_body(x_ref, yt_ref, o_ref):
    x = x_ref[0]
    yt = yt_ref[0]
    xx = jnp.sum(x * x, axis=1, keepdims=True)
    yy = jnp.sum(yt * yt, axis=0, keepdims=True)
    inner = -2.0 * jax.lax.dot_general(
        x, yt, (((1,), (0,)), ((), ())),
        preferred_element_type=jnp.float32)
    d = (xx + inner) + yy
    col = jax.lax.broadcasted_iota(jnp.int32, (BQ, N), 1)
    cols = []
    for _ in range(K):
        j = jnp.argmin(d, axis=1).astype(jnp.int32)[:, None]
        cols.append(j)
        d = jnp.where(col == j, BIG, d)
    o_ref[0] = jnp.concatenate(cols, axis=1)


def _knn_tc(xyz_tc):
    nb = xyz_tc.shape[0]
    yt = jnp.transpose(xyz_tc, (0, 2, 1))
    return pl.pallas_call(
        _knn_tc_body,
        grid=(nb, N // BQ),
        in_specs=[
            pl.BlockSpec((1, BQ, 3), lambda b, q: (b, q, 0)),
            pl.BlockSpec((1, 3, N), lambda b, q: (b, 0, 0)),
        ],
        out_specs=pl.BlockSpec((1, BQ, K), lambda b, q: (b, q, 0)),
        out_shape=jax.ShapeDtypeStruct((nb, N, K), jnp.int32),
    )(xyz_tc, yt)


def kernel(xyz):
    out_sc = _knn_sc(xyz[NB_TC:])
    if NB_TC == 0:
        return out_sc
    out_tc = _knn_tc(xyz[:NB_TC])
    return jnp.concatenate([out_tc, out_sc], axis=0)
